# baseline (speedup 1.0000x reference)
import functools

import jax
import jax.numpy as jnp
from jax import lax
from jax.experimental import pallas as pl
from jax.experimental.pallas import tpu as pltpu
from jax.experimental.pallas import tpu_sc as plsc

_B = 4096
_N = 1000
_LANES = 16
_ROWS_PER_W = 32
_CHUNK = 8
_NCHUNKS = _ROWS_PER_W // _CHUNK


def _remap_body(nc, xt_hbm, map_hbm, out_hbm,
                idx_raw, idx_c, rows_v,
                sem_in0, sem_in1, sem_out0, sem_out1):
    cid = lax.axis_index("c")
    sid = lax.axis_index("s")
    wid = sid * nc + cid
    base = jnp.minimum(wid * _ROWS_PER_W, _N - _ROWS_PER_W)

    pltpu.sync_copy(map_hbm.at[pl.ds(base, _ROWS_PER_W)], idx_raw)
    for v in range(_ROWS_PER_W // _LANES):
        idx_c[pl.ds(v * _LANES, _LANES)] = jnp.minimum(
            idx_raw[pl.ds(v * _LANES, _LANES)], _N - 1)

    sem_in = (sem_in0, sem_in1)
    sem_out = (sem_out0, sem_out1)
    zeros = jnp.zeros((_LANES,), jnp.float32)

    def start_in(c):
        ph = c % 2
        return pltpu.async_copy(
            xt_hbm.at[idx_c.at[pl.ds(c * _CHUNK, _CHUNK)]],
            rows_v.at[ph], sem_in[ph])

    pending_in = {0: start_in(0)}
    pending_out = {}
    for c in range(_NCHUNKS):
        ph = c % 2
        pending_in.pop(c).wait()
        for r in range(_CHUNK):
            bvec = plsc.load_gather(
                idx_raw, [jnp.full((_LANES,), c * _CHUNK + r, jnp.int32)])
            sentinel = lax.reduce_max(bvec, (0,))

            @pl.when(sentinel >= _N)
            def _():
                def zstep(i, _):
                    rows_v[ph, r, pl.ds(i * _LANES, _LANES)] = zeros
                    return 0
                lax.fori_loop(0, _B // _LANES, zstep, 0)
        pending_out[c] = pltpu.async_copy(
            rows_v.at[ph],
            out_hbm.at[pl.ds(base + c * _CHUNK, _CHUNK)], sem_out[ph])
        if c + 1 < _NCHUNKS:
            if c - 1 in pending_out:
                pending_out.pop(c - 1).wait()
            pending_in[c + 1] = start_in(c + 1)

    for c in sorted(pending_out):
        pending_out.pop(c).wait()


def kernel(x, mapping):
    mapping = mapping.astype(jnp.int32)

    info = plsc.get_sparse_core_info()
    nw = info.num_cores * info.num_subcores
    assert nw * _ROWS_PER_W >= _N

    mesh = plsc.VectorSubcoreMesh(core_axis_name="c", subcore_axis_name="s")
    f = pl.kernel(
        functools.partial(_remap_body, info.num_cores),
        out_type=jax.ShapeDtypeStruct((_N, _B), jnp.float32),
        mesh=mesh,
        compiler_params=pltpu.CompilerParams(needs_layout_passes=False),
        scratch_types=[
            pltpu.VMEM((_ROWS_PER_W,), jnp.int32),
            pltpu.VMEM((_ROWS_PER_W,), jnp.int32),
            pltpu.VMEM((2, _CHUNK, _B), jnp.float32),
            pltpu.SemaphoreType.DMA,
            pltpu.SemaphoreType.DMA,
            pltpu.SemaphoreType.DMA,
            pltpu.SemaphoreType.DMA,
        ],
    )
    return f(x.T, mapping).T


# speedup vs baseline: 1.0011x; 1.0011x over previous
import functools

import jax
import jax.numpy as jnp
from jax import lax
from jax.experimental import pallas as pl
from jax.experimental.pallas import tpu as pltpu
from jax.experimental.pallas import tpu_sc as plsc

_B = 4096
_N = 1000
_LANES = 16
_ROWS_PER_W = 32
_CHUNK = 8
_NCHUNKS = _ROWS_PER_W // _CHUNK


def _remap_body(nc, xt_hbm, map_hbm, out_hbm,
                idx_raw, idx_c, rows_v,
                sem_in0, sem_in1, sem_out0, sem_out1):
    cid = lax.axis_index("c")
    sid = lax.axis_index("s")
    wid = sid * nc + cid
    base = jnp.minimum(wid * _ROWS_PER_W, _N - _ROWS_PER_W)

    pltpu.sync_copy(map_hbm.at[pl.ds(base, _ROWS_PER_W)],
                    idx_raw.at[pl.ds(8, _ROWS_PER_W)])
    for v in range(_ROWS_PER_W // _LANES):
        idx_c[pl.ds(v * _LANES, _LANES)] = jnp.minimum(
            idx_raw[pl.ds(8 + v * _LANES, _LANES)], _N - 1)

    sem_in = (sem_in0, sem_in1)
    sem_out = (sem_out0, sem_out1)
    zeros = jnp.zeros((_LANES,), jnp.float32)

    def start_in(c):
        ph = c % 2
        return pltpu.async_copy(
            xt_hbm.at[idx_c.at[pl.ds(c * _CHUNK, _CHUNK)]],
            rows_v.at[ph], sem_in[ph])

    pending_in = {0: start_in(0)}
    pending_out = {}
    for c in range(_NCHUNKS):
        ph = c % 2
        pending_in.pop(c).wait()
        for r in range(_CHUNK):
            bvec = plsc.load_gather(
                idx_raw,
                [jnp.full((_LANES,), 8 + c * _CHUNK + r, jnp.int32)])
            sentinel = lax.reduce_max(bvec, (0,))

            @pl.when(sentinel >= _N)
            def _():
                def zstep(i, _):
                    rows_v[ph, r, pl.ds(i * _LANES, _LANES)] = zeros
                    return 0
                lax.fori_loop(0, _B // _LANES, zstep, 0)
        pending_out[c] = pltpu.async_copy(
            rows_v.at[ph],
            out_hbm.at[pl.ds(base + c * _CHUNK, _CHUNK)], sem_out[ph])
        if c + 1 < _NCHUNKS:
            if c - 1 in pending_out:
                pending_out.pop(c - 1).wait()
            pending_in[c + 1] = start_in(c + 1)

    for c in sorted(pending_out):
        pending_out.pop(c).wait()


def kernel(x, mapping):
    mapping = mapping.astype(jnp.int32)

    info = plsc.get_sparse_core_info()
    nw = info.num_cores * info.num_subcores
    assert nw * _ROWS_PER_W >= _N

    mesh = plsc.VectorSubcoreMesh(core_axis_name="c", subcore_axis_name="s")
    f = pl.kernel(
        functools.partial(_remap_body, info.num_cores),
        out_type=jax.ShapeDtypeStruct((_N, _B), jnp.float32),
        mesh=mesh,
        compiler_params=pltpu.CompilerParams(needs_layout_passes=False),
        scratch_types=[
            pltpu.VMEM((8 + _ROWS_PER_W,), jnp.int32),
            pltpu.VMEM((_ROWS_PER_W,), jnp.int32),
            pltpu.VMEM((2, _CHUNK, _B), jnp.float32),
            pltpu.SemaphoreType.DMA,
            pltpu.SemaphoreType.DMA,
            pltpu.SemaphoreType.DMA,
            pltpu.SemaphoreType.DMA,
        ],
    )
    return f(x.T, mapping).T


# speedup vs baseline: 1.0518x; 1.0507x over previous
import functools

import jax
import jax.numpy as jnp
from jax import lax
from jax.experimental import pallas as pl
from jax.experimental.pallas import tpu as pltpu
from jax.experimental.pallas import tpu_sc as plsc

_B = 4096
_N = 1000
_LANES = 16
_ROWS_PER_W = 32
_CHUNK = 8
_NCHUNKS = _ROWS_PER_W // _CHUNK


def _remap_body(nc, xt_hbm, map_hbm, out_hbm,
                idx_raw, idx_c, rows_v,
                sem_in0, sem_in1, sem_out0, sem_out1):
    cid = lax.axis_index("c")
    sid = lax.axis_index("s")
    wid = sid * nc + cid
    base = jnp.minimum(wid * _ROWS_PER_W, _N - _ROWS_PER_W)

    pltpu.sync_copy(map_hbm.at[pl.ds(base, _ROWS_PER_W)],
                    idx_raw.at[pl.ds(8, _ROWS_PER_W)])
    for v in range(_ROWS_PER_W // _LANES):
        idx_c[pl.ds(v * _LANES, _LANES)] = jnp.minimum(
            idx_raw[pl.ds(8 + v * _LANES, _LANES)], _N - 1)

    sem_in = (sem_in0, sem_in1)
    sem_out = (sem_out0, sem_out1)
    zeros = jnp.zeros((_LANES,), jnp.float32)

    def start_in(c):
        return pltpu.async_copy(
            xt_hbm.at[idx_c.at[pl.ds(c * _CHUNK, _CHUNK)]],
            rows_v.at[c % 3], sem_in[c % 2])

    pending_in = {0: start_in(0), 1: start_in(1)}
    pending_out = {}
    for c in range(_NCHUNKS):
        ph = c % 3
        pending_in.pop(c).wait()
        for r in range(_CHUNK):
            bvec = plsc.load_gather(
                idx_raw,
                [jnp.full((_LANES,), 8 + c * _CHUNK + r, jnp.int32)])
            sentinel = lax.reduce_max(bvec, (0,))

            @pl.when(sentinel >= _N)
            def _():
                def zstep(i, _):
                    rows_v[ph, r, pl.ds(i * _LANES, _LANES)] = zeros
                    return 0
                lax.fori_loop(0, _B // _LANES, zstep, 0)
        pending_out[c] = pltpu.async_copy(
            rows_v.at[ph],
            out_hbm.at[pl.ds(base + c * _CHUNK, _CHUNK)], sem_out[c % 2])
        if c + 2 < _NCHUNKS:
            if c - 1 in pending_out:
                pending_out.pop(c - 1).wait()
            pending_in[c + 2] = start_in(c + 2)

    for c in sorted(pending_out):
        pending_out.pop(c).wait()


def kernel(x, mapping):
    mapping = mapping.astype(jnp.int32)

    info = plsc.get_sparse_core_info()
    nw = info.num_cores * info.num_subcores
    assert nw * _ROWS_PER_W >= _N

    mesh = plsc.VectorSubcoreMesh(core_axis_name="c", subcore_axis_name="s")
    f = pl.kernel(
        functools.partial(_remap_body, info.num_cores),
        out_type=jax.ShapeDtypeStruct((_N, _B), jnp.float32),
        mesh=mesh,
        compiler_params=pltpu.CompilerParams(needs_layout_passes=False),
        scratch_types=[
            pltpu.VMEM((8 + _ROWS_PER_W,), jnp.int32),
            pltpu.VMEM((_ROWS_PER_W,), jnp.int32),
            pltpu.VMEM((3, _CHUNK, _B), jnp.float32),
            pltpu.SemaphoreType.DMA,
            pltpu.SemaphoreType.DMA,
            pltpu.SemaphoreType.DMA,
            pltpu.SemaphoreType.DMA,
        ],
    )
    return f(x.T, mapping).T


# speedup vs baseline: 1.0690x; 1.0163x over previous
import functools

import jax
import jax.numpy as jnp
from jax import lax
from jax.experimental import pallas as pl
from jax.experimental.pallas import tpu as pltpu
from jax.experimental.pallas import tpu_sc as plsc

_B = 4096
_N = 1000
_LANES = 16
_ROWS_PER_W = 32
_CHUNK = 8
_NCHUNKS = _ROWS_PER_W // _CHUNK


def _remap_body(nc, xt_hbm, map_hbm, out_hbm,
                idx_raw, idx_c, rows_v,
                sem_in0, sem_in1, sem_out0, sem_out1):
    cid = lax.axis_index("c")
    sid = lax.axis_index("s")
    wid = sid * nc + cid
    base = jnp.minimum(wid * _ROWS_PER_W, _N - _ROWS_PER_W)

    pltpu.sync_copy(map_hbm.at[pl.ds(base, _ROWS_PER_W)],
                    idx_raw.at[pl.ds(8, _ROWS_PER_W)])
    for v in range(_ROWS_PER_W // _LANES):
        idx_c[pl.ds(v * _LANES, _LANES)] = jnp.minimum(
            idx_raw[pl.ds(8 + v * _LANES, _LANES)], _N - 1)

    sem_in = (sem_in0, sem_in1)
    sem_out = (sem_out0, sem_out1)
    zeros = jnp.zeros((_LANES,), jnp.float32)

    def start_in(c):
        return pltpu.async_copy(
            xt_hbm.at[idx_c.at[pl.ds(c * _CHUNK, _CHUNK)]],
            rows_v.at[c % 3], sem_in[c % 2])

    pending_in = {0: start_in(0), 1: start_in(1)}
    pending_out = {}

    sentinels = [
        lax.reduce_max(
            plsc.load_gather(
                idx_raw, [jnp.full((_LANES,), 8 + i, jnp.int32)]),
            (0,))
        for i in range(_ROWS_PER_W)
    ]

    for c in range(_NCHUNKS):
        ph = c % 3
        pending_in.pop(c).wait()
        for r in range(_CHUNK):
            @pl.when(sentinels[c * _CHUNK + r] >= _N)
            def _():
                def zstep(i, _):
                    rows_v[ph, r, pl.ds(i * _LANES, _LANES)] = zeros
                    return 0
                lax.fori_loop(0, _B // _LANES, zstep, 0)
        pending_out[c] = pltpu.async_copy(
            rows_v.at[ph],
            out_hbm.at[pl.ds(base + c * _CHUNK, _CHUNK)], sem_out[c % 2])
        if c + 2 < _NCHUNKS:
            if c - 1 in pending_out:
                pending_out.pop(c - 1).wait()
            pending_in[c + 2] = start_in(c + 2)

    for c in sorted(pending_out):
        pending_out.pop(c).wait()


def kernel(x, mapping):
    mapping = mapping.astype(jnp.int32)

    info = plsc.get_sparse_core_info()
    nw = info.num_cores * info.num_subcores
    assert nw * _ROWS_PER_W >= _N

    mesh = plsc.VectorSubcoreMesh(core_axis_name="c", subcore_axis_name="s")
    f = pl.kernel(
        functools.partial(_remap_body, info.num_cores),
        out_type=jax.ShapeDtypeStruct((_N, _B), jnp.float32),
        mesh=mesh,
        compiler_params=pltpu.CompilerParams(needs_layout_passes=False),
        scratch_types=[
            pltpu.VMEM((8 + _ROWS_PER_W,), jnp.int32),
            pltpu.VMEM((_ROWS_PER_W,), jnp.int32),
            pltpu.VMEM((3, _CHUNK, _B), jnp.float32),
            pltpu.SemaphoreType.DMA,
            pltpu.SemaphoreType.DMA,
            pltpu.SemaphoreType.DMA,
            pltpu.SemaphoreType.DMA,
        ],
    )
    return f(x.T, mapping).T
